# baseline (speedup 1.0000x reference)
import functools

import jax
import jax.numpy as jnp
from jax import lax
from jax.experimental import pallas as pl
from jax.experimental.pallas import tpu as pltpu
from jax.experimental.pallas import tpu_sc as plsc

_LANES = 16
_NWORKERS = 32


def _softplus(v):
    return jnp.maximum(v, 0.0) + jnp.log(1.0 + jnp.exp(-jnp.abs(v)))


def _pos_parts_body(fh_ref, fw_ref, ph_ref, pw_ref, W_ref, b_ref,
                    hpart_ref, wpart_ref):
    n = fh_ref.shape[1]
    H = hpart_ref.shape[0]
    W = wpart_ref.shape[0]

    fh = _softplus(fh_ref[...]) * 10.0
    fw = _softplus(fw_ref[...]) * 10.0

    h_pos = lax.broadcasted_iota(jnp.int32, (H, 1), 0).astype(jnp.float32) * (
        1.0 / H)
    w_pos = lax.broadcasted_iota(jnp.int32, (W, 1), 0).astype(jnp.float32) * (
        1.0 / W)
    h_ang = h_pos * fh + ph_ref[...]
    w_ang = w_pos * fw + pw_ref[...]
    h_enc = jnp.concatenate([jnp.sin(h_ang), jnp.cos(h_ang)], axis=1)
    w_enc = jnp.concatenate([jnp.sin(w_ang), jnp.cos(w_ang)], axis=1)

    Wm = W_ref[...]
    hpart_ref[...] = jnp.dot(h_enc, Wm[: 2 * n, :],
                             preferred_element_type=jnp.float32)
    wpart_ref[...] = (jnp.dot(w_enc, Wm[2 * n:, :],
                              preferred_element_type=jnp.float32)
                      + b_ref[...])


def _pos_parts(freq_h, freq_w, phase_h, phase_w, pos_W, pos_b, H, W):
    D = pos_W.shape[1]
    return pl.pallas_call(
        _pos_parts_body,
        out_shape=[
            jax.ShapeDtypeStruct((H, D), jnp.float32),
            jax.ShapeDtypeStruct((W, D), jnp.float32),
        ],
    )(freq_h.reshape(1, -1), freq_w.reshape(1, -1),
      phase_h.reshape(1, -1), phase_w.reshape(1, -1),
      pos_W, pos_b.reshape(1, -1))


def _rsqrt_vec(ms):
    i = lax.bitcast_convert_type(ms, jnp.int32)
    i = jnp.int32(0x5F3759DF) - lax.shift_right_logical(i, 1)
    y = lax.bitcast_convert_type(i, jnp.float32)
    half = ms * 0.5
    for _ in range(3):
        y = y * (1.5 - half * y * y)
    return y


def _make_sc_kernel(B, H, W, D, V):
    NT = B * H
    TPW = NT // _NWORKERS
    NG = W // _LANES
    NPAIR = NG // 2
    ROW = W * D
    GRP = _LANES * D
    NJ = D // _LANES

    mesh = plsc.VectorSubcoreMesh(core_axis_name="c", subcore_axis_name="s")

    @functools.partial(
        pl.kernel,
        out_type=jax.ShapeDtypeStruct((NT, ROW), jnp.float32),
        mesh=mesh,
        compiler_params=pltpu.CompilerParams(needs_layout_passes=False),
        scratch_types=[
            pltpu.VMEM((V * D,), jnp.float32),
            pltpu.VMEM((W * D,), jnp.float32),
            pltpu.VMEM((D,), jnp.float32),
            pltpu.VMEM((2 * W,), jnp.float32),
            pltpu.VMEM((2 * D,), jnp.float32),
            pltpu.VMEM((GRP,), jnp.float32),
            pltpu.VMEM((GRP,), jnp.float32),
            pltpu.VMEM((2 * _LANES,), jnp.float32),
            pltpu.VMEM((_LANES,), jnp.int32),
            pltpu.VMEM((_LANES,), jnp.float32),
            pltpu.SemaphoreType.DMA,
            pltpu.SemaphoreType.DMA,
            pltpu.SemaphoreType.DMA,
        ],
    )
    def sc_kernel(x_hbm, tab_hbm, w_hbm, h_hbm, rms_hbm, out_hbm,
                  tab_v, w_v, rms_v, x_v, h_v, tmp0, tmp1, sbuf, ibuf, rbuf,
                  sem0, sem1, semx):
        wid = lax.axis_index("s") * 2 + lax.axis_index("c")
        t0 = wid * TPW

        pltpu.sync_copy(tab_hbm, tab_v)
        pltpu.sync_copy(w_hbm, w_v)
        pltpu.sync_copy(rms_hbm, rms_v)
        pltpu.sync_copy(x_hbm.at[t0], x_v.at[pl.ds(0, W)])
        pltpu.sync_copy(h_hbm.at[t0 % H], h_v.at[pl.ds(0, D)])

        iota = lax.iota(jnp.int32, _LANES)
        coffs = [iota + 16 * j for j in range(NJ)]
        rmsw = [rms_v[pl.ds(16 * j, _LANES)] for j in range(NJ)]
        lane15 = iota == 15

        def task_body(i, _):
            t = t0 + i
            pi = lax.rem(i, 2)

            @pl.when(i > 0)
            def _():
                pltpu.make_async_copy(
                    x_hbm.at[t], x_v.at[pl.ds(pi * W, W)], semx).wait()
                pltpu.make_async_copy(
                    h_hbm.at[t % H], h_v.at[pl.ds(pi * D, D)], semx).wait()

            @pl.when(i + 1 < TPW)
            def _():
                tn = t + 1
                po = (1 - pi)
                pltpu.async_copy(x_hbm.at[tn], x_v.at[pl.ds(po * W, W)], semx)
                pltpu.async_copy(h_hbm.at[lax.rem(tn, H)],
                                 h_v.at[pl.ds(po * D, D)], semx)

            hrow = [h_v[pl.ds(pi * D + 16 * j, _LANES)] for j in range(NJ)]

            def pair_body(gp, _):
                for sub, (tmp, sem) in enumerate(((tmp0, sem0), (tmp1, sem1))):
                    g = gp * 2 + sub
                    gg = i * NG + g

                    @pl.when(gg >= 2)
                    def _():
                        pltpu.make_async_copy(
                            tmp, out_hbm.at[t, pl.ds(0, GRP)], sem).wait()

                    xg = x_v[pl.ds(pi * W + g * _LANES, _LANES)]
                    iv = jnp.clip(xg * 255.0, 0.0, 255.0).astype(jnp.int32)
                    ibuf[...] = iv * D

                    @plsc.parallel_loop(0, _LANES, unroll=16)
                    def _(p):
                        pv = jnp.broadcast_to(p, (_LANES,))
                        tbv = plsc.load_gather(ibuf, [pv])
                        wb = (g * _LANES + p) * D
                        acc = None
                        for j in range(NJ):
                            ev = plsc.load_gather(tab_v, [tbv + coffs[j]])
                            wv = w_v[pl.ds(wb + 16 * j, _LANES)]
                            val = ev + wv + hrow[j]
                            tmp[pl.ds(p * D + 16 * j, _LANES)] = val
                            sq = val * val
                            acc = sq if acc is None else acc + sq
                        cs = plsc.cumsum(acc)
                        plsc.store_compressed(sbuf.at[pl.ds(p, _LANES)], cs,
                                              mask=lane15)

                    sv = sbuf[pl.ds(0, _LANES)]
                    rbuf[...] = _rsqrt_vec(sv * (1.0 / D) + 1e-6)

                    @plsc.parallel_loop(0, _LANES, unroll=16)
                    def _(p):
                        pv = jnp.broadcast_to(p, (_LANES,))
                        rsv = plsc.load_gather(rbuf, [pv])
                        for j in range(NJ):
                            sl = pl.ds(p * D + 16 * j, _LANES)
                            tmp[sl] = tmp[sl] * rsv * rmsw[j]

                    pltpu.async_copy(
                        tmp, out_hbm.at[t, pl.ds(g * GRP, GRP)], sem)

            lax.fori_loop(0, NPAIR, pair_body, None)

        lax.fori_loop(0, TPW, task_body, None)

        pltpu.make_async_copy(tmp0, out_hbm.at[0, pl.ds(0, GRP)], sem0).wait()
        pltpu.make_async_copy(tmp1, out_hbm.at[0, pl.ds(0, GRP)], sem1).wait()

    return sc_kernel


def kernel(x, pixel_embed, freq_h, freq_w, phase_h, phase_w, pos_W, pos_b,
           rms_w):
    B, H, W = x.shape
    V, D = pixel_embed.shape

    h_part, w_part = _pos_parts(freq_h, freq_w, phase_h, phase_w,
                                pos_W, pos_b, H, W)

    sc = _make_sc_kernel(B, H, W, D, V)
    out = sc(x.reshape(B * H, W), pixel_embed.reshape(-1),
             w_part.reshape(-1), h_part, rms_w)
    return out.reshape(B, H * W, D)


# speedup vs baseline: 1.1693x; 1.1693x over previous
import functools

import jax
import jax.numpy as jnp
from jax import lax
from jax.experimental import pallas as pl
from jax.experimental.pallas import tpu as pltpu
from jax.experimental.pallas import tpu_sc as plsc

_LANES = 16
_NWORKERS = 32


def _softplus(v):
    return jnp.maximum(v, 0.0) + jnp.log(1.0 + jnp.exp(-jnp.abs(v)))


def _pos_parts_body(fh_ref, fw_ref, ph_ref, pw_ref, W_ref, b_ref,
                    hpart_ref, wpart_ref):
    n = fh_ref.shape[1]
    H = hpart_ref.shape[0]
    W = wpart_ref.shape[0]

    fh = _softplus(fh_ref[...]) * 10.0
    fw = _softplus(fw_ref[...]) * 10.0

    h_pos = lax.broadcasted_iota(jnp.int32, (H, 1), 0).astype(jnp.float32) * (
        1.0 / H)
    w_pos = lax.broadcasted_iota(jnp.int32, (W, 1), 0).astype(jnp.float32) * (
        1.0 / W)
    h_ang = h_pos * fh + ph_ref[...]
    w_ang = w_pos * fw + pw_ref[...]
    h_enc = jnp.concatenate([jnp.sin(h_ang), jnp.cos(h_ang)], axis=1)
    w_enc = jnp.concatenate([jnp.sin(w_ang), jnp.cos(w_ang)], axis=1)

    Wm = W_ref[...]
    hpart_ref[...] = jnp.dot(h_enc, Wm[: 2 * n, :],
                             preferred_element_type=jnp.float32)
    wpart_ref[...] = (jnp.dot(w_enc, Wm[2 * n:, :],
                              preferred_element_type=jnp.float32)
                      + b_ref[...])


def _pos_parts(freq_h, freq_w, phase_h, phase_w, pos_W, pos_b, H, W):
    D = pos_W.shape[1]
    return pl.pallas_call(
        _pos_parts_body,
        out_shape=[
            jax.ShapeDtypeStruct((H, D), jnp.float32),
            jax.ShapeDtypeStruct((W, D), jnp.float32),
        ],
    )(freq_h.reshape(1, -1), freq_w.reshape(1, -1),
      phase_h.reshape(1, -1), phase_w.reshape(1, -1),
      pos_W, pos_b.reshape(1, -1))


def _rsqrt_vec(ms):
    i = lax.bitcast_convert_type(ms, jnp.int32)
    i = jnp.int32(0x5F3759DF) - lax.shift_right_logical(i, 1)
    y = lax.bitcast_convert_type(i, jnp.float32)
    half = ms * 0.5
    for _ in range(3):
        y = y * (1.5 - half * y * y)
    return y


def _make_sc_kernel(B, H, W, D, V):
    TPW = H // _NWORKERS
    NG = W // _LANES
    NPAIR = NG // 2
    ROW = W * D
    GRP = _LANES * D
    NJ = D // _LANES
    BW = B * W

    mesh = plsc.VectorSubcoreMesh(core_axis_name="c", subcore_axis_name="s")

    @functools.partial(
        pl.kernel,
        out_type=jax.ShapeDtypeStruct((B * H, ROW), jnp.float32),
        mesh=mesh,
        compiler_params=pltpu.CompilerParams(needs_layout_passes=False),
        scratch_types=[
            pltpu.VMEM((V * D,), jnp.float32),
            pltpu.VMEM((W * D,), jnp.float32),
            pltpu.VMEM((D,), jnp.float32),
            pltpu.VMEM((2 * BW,), jnp.float32),
            pltpu.VMEM((2 * D,), jnp.float32),
            pltpu.VMEM((B * GRP,), jnp.float32),
            pltpu.VMEM((B * GRP,), jnp.float32),
            pltpu.VMEM((B * _LANES + _LANES,), jnp.float32),
            pltpu.VMEM((B * _LANES,), jnp.int32),
            pltpu.VMEM((B * _LANES,), jnp.float32),
            pltpu.SemaphoreType.DMA,
            pltpu.SemaphoreType.DMA,
            pltpu.SemaphoreType.DMA,
        ],
    )
    def sc_kernel(x_hbm, tab_hbm, w_hbm, h_hbm, rms_hbm, out_hbm,
                  tab_v, w_v, rms_v, x_v, h_v, tmp0, tmp1, sbuf, ibuf, rbuf,
                  sem0, sem1, semx):
        wid = lax.axis_index("s") * 2 + lax.axis_index("c")
        t0 = wid * TPW

        pltpu.sync_copy(tab_hbm, tab_v)
        pltpu.sync_copy(w_hbm, w_v)
        pltpu.sync_copy(rms_hbm, rms_v)
        pltpu.sync_copy(x_hbm.at[t0], x_v.at[pl.ds(0, BW)])
        pltpu.sync_copy(h_hbm.at[t0], h_v.at[pl.ds(0, D)])

        iota = lax.iota(jnp.int32, _LANES)
        coffs = [iota + 16 * j for j in range(NJ)]
        rmsw = [rms_v[pl.ds(16 * j, _LANES)] for j in range(NJ)]
        lane15 = iota == 15

        def task_body(i, _):
            t = t0 + i
            pi = lax.rem(i, 2)

            @pl.when(i > 0)
            def _():
                pltpu.make_async_copy(
                    x_hbm.at[t], x_v.at[pl.ds(pi * BW, BW)], semx).wait()
                pltpu.make_async_copy(
                    h_hbm.at[t], h_v.at[pl.ds(pi * D, D)], semx).wait()

            @pl.when(i + 1 < TPW)
            def _():
                tn = t + 1
                po = (1 - pi)
                pltpu.async_copy(x_hbm.at[tn], x_v.at[pl.ds(po * BW, BW)],
                                 semx)
                pltpu.async_copy(h_hbm.at[tn], h_v.at[pl.ds(po * D, D)],
                                 semx)

            hrow = [h_v[pl.ds(pi * D + 16 * j, _LANES)] for j in range(NJ)]

            def pair_body(gp, _):
                for sub, (tmp, sem) in enumerate(((tmp0, sem0), (tmp1, sem1))):
                    g = gp * 2 + sub
                    gg = i * NG + g

                    @pl.when(gg >= 2)
                    def _():
                        for b in range(B):
                            pltpu.make_async_copy(
                                tmp.at[pl.ds(b * GRP, GRP)],
                                out_hbm.at[t, pl.ds(0, GRP)], sem).wait()

                    for b in range(B):
                        xg = x_v[pl.ds(pi * BW + b * W + g * _LANES, _LANES)]
                        iv = jnp.clip(xg * 255.0, 0.0,
                                      255.0).astype(jnp.int32)
                        ibuf[pl.ds(b * _LANES, _LANES)] = iv * D

                    @plsc.parallel_loop(0, B * _LANES, unroll=8)
                    def _(i2):
                        p = jnp.bitwise_and(i2, _LANES - 1)
                        base = i2 * D
                        wb = (g * _LANES + p) * D
                        tbv = plsc.load_gather(
                            ibuf, [jnp.broadcast_to(i2, (_LANES,))])
                        evs = [plsc.load_gather(tab_v, [tbv + coffs[j]])
                               for j in range(NJ)]
                        vals = [evs[j] + w_v[pl.ds(wb + 16 * j, _LANES)]
                                + hrow[j] for j in range(NJ)]
                        acc = None
                        for j in range(NJ):
                            sq = vals[j] * vals[j]
                            acc = sq if acc is None else acc + sq
                        cs = plsc.cumsum(acc)
                        for j in range(NJ):
                            tmp[pl.ds(base + 16 * j, _LANES)] = vals[j]
                        plsc.store_compressed(
                            sbuf.at[pl.ds(i2, _LANES)], cs, mask=lane15)

                    for b in range(B):
                        sv = sbuf[pl.ds(b * _LANES, _LANES)]
                        rbuf[pl.ds(b * _LANES, _LANES)] = _rsqrt_vec(
                            sv * (1.0 / D) + 1e-6)

                    @plsc.parallel_loop(0, B * _LANES, unroll=8)
                    def _(i2):
                        base = i2 * D
                        rsv = plsc.load_gather(
                            rbuf, [jnp.broadcast_to(i2, (_LANES,))])
                        vs = [tmp[pl.ds(base + 16 * j, _LANES)]
                              for j in range(NJ)]
                        for j in range(NJ):
                            tmp[pl.ds(base + 16 * j, _LANES)] = (
                                vs[j] * rsv * rmsw[j])

                    for b in range(B):
                        pltpu.async_copy(
                            tmp.at[pl.ds(b * GRP, GRP)],
                            out_hbm.at[b * H + t, pl.ds(g * GRP, GRP)], sem)

            lax.fori_loop(0, NPAIR, pair_body, None)

        lax.fori_loop(0, TPW, task_body, None)

        for b in range(B):
            pltpu.make_async_copy(tmp0.at[pl.ds(b * GRP, GRP)],
                                  out_hbm.at[0, pl.ds(0, GRP)], sem0).wait()
            pltpu.make_async_copy(tmp1.at[pl.ds(b * GRP, GRP)],
                                  out_hbm.at[0, pl.ds(0, GRP)], sem1).wait()

    return sc_kernel


def kernel(x, pixel_embed, freq_h, freq_w, phase_h, phase_w, pos_W, pos_b,
           rms_w):
    B, H, W = x.shape
    V, D = pixel_embed.shape

    h_part, w_part = _pos_parts(freq_h, freq_w, phase_h, phase_w,
                                pos_W, pos_b, H, W)

    sc = _make_sc_kernel(B, H, W, D, V)
    out = sc(x.transpose(1, 0, 2).reshape(H, B * W), pixel_embed.reshape(-1),
             w_part.reshape(-1), h_part, rms_w)
    return out.reshape(B, H * W, D)


# speedup vs baseline: 1.1990x; 1.0254x over previous
import functools

import jax
import jax.numpy as jnp
from jax import lax
from jax.experimental import pallas as pl
from jax.experimental.pallas import tpu as pltpu
from jax.experimental.pallas import tpu_sc as plsc

_LANES = 16
_NWORKERS = 32


def _softplus(v):
    return jnp.maximum(v, 0.0) + jnp.log(1.0 + jnp.exp(-jnp.abs(v)))


def _pos_parts_body(fh_ref, fw_ref, ph_ref, pw_ref, W_ref, b_ref,
                    hpart_ref, wpart_ref):
    n = fh_ref.shape[1]
    H = hpart_ref.shape[0]
    W = wpart_ref.shape[0]

    fh = _softplus(fh_ref[...]) * 10.0
    fw = _softplus(fw_ref[...]) * 10.0

    h_pos = lax.broadcasted_iota(jnp.int32, (H, 1), 0).astype(jnp.float32) * (
        1.0 / H)
    w_pos = lax.broadcasted_iota(jnp.int32, (W, 1), 0).astype(jnp.float32) * (
        1.0 / W)
    h_ang = h_pos * fh + ph_ref[...]
    w_ang = w_pos * fw + pw_ref[...]
    h_enc = jnp.concatenate([jnp.sin(h_ang), jnp.cos(h_ang)], axis=1)
    w_enc = jnp.concatenate([jnp.sin(w_ang), jnp.cos(w_ang)], axis=1)

    Wm = W_ref[...]
    hpart_ref[...] = jnp.dot(h_enc, Wm[: 2 * n, :],
                             preferred_element_type=jnp.float32)
    wpart_ref[...] = (jnp.dot(w_enc, Wm[2 * n:, :],
                              preferred_element_type=jnp.float32)
                      + b_ref[...])


def _pos_parts(freq_h, freq_w, phase_h, phase_w, pos_W, pos_b, H, W):
    D = pos_W.shape[1]
    return pl.pallas_call(
        _pos_parts_body,
        out_shape=[
            jax.ShapeDtypeStruct((H, D), jnp.float32),
            jax.ShapeDtypeStruct((W, D), jnp.float32),
        ],
    )(freq_h.reshape(1, -1), freq_w.reshape(1, -1),
      phase_h.reshape(1, -1), phase_w.reshape(1, -1),
      pos_W, pos_b.reshape(1, -1))


def _rsqrt_vec(ms):
    i = lax.bitcast_convert_type(ms, jnp.int32)
    i = jnp.int32(0x5F3759DF) - lax.shift_right_logical(i, 1)
    y = lax.bitcast_convert_type(i, jnp.float32)
    half = ms * 0.5
    for _ in range(3):
        y = y * (1.5 - half * y * y)
    return y


def _make_sc_kernel(B, H, W, D, V):
    TPW = H // _NWORKERS
    NG = W // _LANES
    NPAIR = NG // 2
    ROW = W * D
    GRP = _LANES * D
    NJ = D // _LANES
    BW = B * W

    mesh = plsc.VectorSubcoreMesh(core_axis_name="c", subcore_axis_name="s")

    @functools.partial(
        pl.kernel,
        out_type=jax.ShapeDtypeStruct((B * H, ROW), jnp.float32),
        mesh=mesh,
        compiler_params=pltpu.CompilerParams(needs_layout_passes=False),
        scratch_types=[
            pltpu.VMEM((V * D,), jnp.float32),
            pltpu.VMEM((W * D,), jnp.float32),
            pltpu.VMEM((D,), jnp.float32),
            pltpu.VMEM((2 * BW,), jnp.float32),
            pltpu.VMEM((2 * D,), jnp.float32),
            pltpu.VMEM((B * GRP,), jnp.float32),
            pltpu.VMEM((B * GRP,), jnp.float32),
            pltpu.VMEM((B * _LANES,), jnp.int32),
            pltpu.SemaphoreType.DMA,
            pltpu.SemaphoreType.DMA,
            pltpu.SemaphoreType.DMA,
        ],
    )
    def sc_kernel(x_hbm, tab_hbm, w_hbm, h_hbm, rms_hbm, out_hbm,
                  tab_v, w_v, rms_v, x_v, h_v, tmp0, tmp1, ibuf,
                  sem0, sem1, semx):
        wid = lax.axis_index("s") * 2 + lax.axis_index("c")
        t0 = wid * TPW

        pltpu.sync_copy(tab_hbm, tab_v)
        pltpu.sync_copy(w_hbm, w_v)
        pltpu.sync_copy(rms_hbm, rms_v)
        pltpu.sync_copy(x_hbm.at[t0], x_v.at[pl.ds(0, BW)])
        pltpu.sync_copy(h_hbm.at[t0], h_v.at[pl.ds(0, D)])

        iota = lax.iota(jnp.int32, _LANES)
        rmsw = [rms_v[pl.ds(16 * j, _LANES)] for j in range(NJ)]

        def task_body(i, _):
            t = t0 + i
            pi = lax.rem(i, 2)

            @pl.when(i > 0)
            def _():
                pltpu.make_async_copy(
                    x_hbm.at[t], x_v.at[pl.ds(pi * BW, BW)], semx).wait()
                pltpu.make_async_copy(
                    h_hbm.at[t], h_v.at[pl.ds(pi * D, D)], semx).wait()

            @pl.when(i + 1 < TPW)
            def _():
                tn = t + 1
                po = (1 - pi)
                pltpu.async_copy(x_hbm.at[tn], x_v.at[pl.ds(po * BW, BW)],
                                 semx)
                pltpu.async_copy(h_hbm.at[tn], h_v.at[pl.ds(po * D, D)],
                                 semx)

            hrow = [h_v[pl.ds(pi * D + 16 * j, _LANES)] for j in range(NJ)]

            def pair_body(gp, _):
                for sub, (tmp, sem) in enumerate(((tmp0, sem0), (tmp1, sem1))):
                    g = gp * 2 + sub
                    gg = i * NG + g

                    @pl.when(gg >= 2)
                    def _():
                        for b in range(B):
                            pltpu.make_async_copy(
                                tmp.at[pl.ds(b * GRP, GRP)],
                                out_hbm.at[t, pl.ds(0, GRP)], sem).wait()

                    for b in range(B):
                        xg = x_v[pl.ds(pi * BW + b * W + g * _LANES, _LANES)]
                        iv = jnp.clip(xg * 255.0, 0.0,
                                      255.0).astype(jnp.int32)
                        ibuf[pl.ds(b * _LANES, _LANES)] = iv * D

                    @plsc.parallel_loop(0, B * _LANES, unroll=2)
                    def _(i2):
                        p = jnp.bitwise_and(i2, _LANES - 1)
                        base = i2 * D
                        wb = (g * _LANES + p) * D
                        tbv = plsc.load_gather(
                            ibuf, [jnp.broadcast_to(i2, (_LANES,))]) + iota
                        evs = [
                            plsc.load_gather(
                                tab_v.at[pl.ds(16 * j, V * D - 16 * j)],
                                [tbv])
                            for j in range(NJ)]
                        vals = [evs[j] + w_v[pl.ds(wb + 16 * j, _LANES)]
                                + hrow[j] for j in range(NJ)]
                        acc = None
                        for j in range(NJ):
                            sq = vals[j] * vals[j]
                            acc = sq if acc is None else acc + sq
                        ms = jnp.sum(acc) * (1.0 / D) + 1e-6
                        r = _rsqrt_vec(ms)
                        rv = jnp.broadcast_to(r, (_LANES,))
                        for j in range(NJ):
                            tmp[pl.ds(base + 16 * j, _LANES)] = (
                                vals[j] * (rv * rmsw[j]))

                    for b in range(B):
                        pltpu.async_copy(
                            tmp.at[pl.ds(b * GRP, GRP)],
                            out_hbm.at[b * H + t, pl.ds(g * GRP, GRP)], sem)

            lax.fori_loop(0, NPAIR, pair_body, None)

        lax.fori_loop(0, TPW, task_body, None)

        for b in range(B):
            pltpu.make_async_copy(tmp0.at[pl.ds(b * GRP, GRP)],
                                  out_hbm.at[0, pl.ds(0, GRP)], sem0).wait()
            pltpu.make_async_copy(tmp1.at[pl.ds(b * GRP, GRP)],
                                  out_hbm.at[0, pl.ds(0, GRP)], sem1).wait()

    return sc_kernel


def kernel(x, pixel_embed, freq_h, freq_w, phase_h, phase_w, pos_W, pos_b,
           rms_w):
    B, H, W = x.shape
    V, D = pixel_embed.shape

    h_part, w_part = _pos_parts(freq_h, freq_w, phase_h, phase_w,
                                pos_W, pos_b, H, W)

    sc = _make_sc_kernel(B, H, W, D, V)
    out = sc(x.transpose(1, 0, 2).reshape(H, B * W), pixel_embed.reshape(-1),
             w_part.reshape(-1), h_part, rms_w)
    return out.reshape(B, H * W, D)
